# baseline (speedup 1.0000x reference)
import jax
import jax.numpy as jnp
from jax import lax
from jax.experimental import pallas as pl
from jax.experimental.pallas import tpu as pltpu

N = 10000
E = 320000
IN_DIM = 128
HID = 128
OUT = 64

NPAD = 10240
CHKE = 16000
NCHKS = E // CHKE
BLK = 256
GRID = NPAD // BLK


def _deg_body(src_ref, dst_ref, ds_ref, dd_ref, acs_ref, acd_ref):
    i = pl.program_id(0)

    @pl.when(i == 0)
    def _():
        def zero(n, _):
            acs_ref[n] = 0.0
            acd_ref[n] = 0.0
            return 0
        lax.fori_loop(0, NPAD, zero, 0)

    def body(e, _):
        s = src_ref[0, 0, e]
        d = dst_ref[0, 0, e]
        acs_ref[s] += 1.0
        acd_ref[d] += 1.0
        return 0

    lax.fori_loop(0, CHKE, body, 0)

    @pl.when(i == NCHKS - 1)
    def _():
        def emit(n, _):
            ds_ref[n] = acs_ref[n]
            dd_ref[n] = acd_ref[n]
            return 0
        lax.fori_loop(0, NPAD, emit, 0)


_deg = pl.pallas_call(
    _deg_body,
    grid=(NCHKS,),
    in_specs=[pl.BlockSpec((1, 1, CHKE), lambda i: (i, 0, 0),
                           memory_space=pltpu.SMEM),
              pl.BlockSpec((1, 1, CHKE), lambda i: (i, 0, 0),
                           memory_space=pltpu.SMEM)],
    out_specs=[pl.BlockSpec((NPAD,), lambda i: (0,),
                            memory_space=pltpu.SMEM),
               pl.BlockSpec((NPAD,), lambda i: (0,),
                            memory_space=pltpu.SMEM)],
    out_shape=[jax.ShapeDtypeStruct((NPAD,), jnp.float32),
               jax.ShapeDtypeStruct((NPAD,), jnp.float32)],
    scratch_shapes=[pltpu.SMEM((NPAD,), jnp.float32),
                    pltpu.SMEM((NPAD,), jnp.float32)],
)


def _scat_body(src_ref, dst_ref, t_ref, agg_ref):
    i = pl.program_id(0)

    @pl.when(i == 0)
    def _():
        agg_ref[...] = jnp.zeros((NPAD, HID), jnp.float32)

    def body(e, _):
        s = src_ref[0, 0, e]
        d = dst_ref[0, 0, e]
        agg_ref[pl.ds(d, 1), :] += t_ref[pl.ds(s, 1), :]
        return 0

    lax.fori_loop(0, CHKE, body, 0)


_scat = pl.pallas_call(
    _scat_body,
    grid=(NCHKS,),
    in_specs=[pl.BlockSpec((1, 1, CHKE), lambda i: (i, 0, 0),
                           memory_space=pltpu.SMEM),
              pl.BlockSpec((1, 1, CHKE), lambda i: (i, 0, 0),
                           memory_space=pltpu.SMEM),
              pl.BlockSpec((NPAD, HID), lambda i: (0, 0))],
    out_specs=pl.BlockSpec((NPAD, HID), lambda i: (0, 0)),
    out_shape=jax.ShapeDtypeStruct((NPAD, HID), jnp.float32),
)


def _norms(ds, dd):
    ns = lax.rsqrt(jnp.maximum(ds, 1.0))
    nd = lax.rsqrt(jnp.maximum(dd, 1.0))
    return ns, nd


def _ka_body(x_ref, w_ref, ds_ref, dd_ref, t0_ref, sq_ref):
    out = jnp.dot(x_ref[...], w_ref[...], preferred_element_type=jnp.float32)
    ns, _ = _norms(ds_ref[...], dd_ref[...])
    t0_ref[...] = out[:, :HID] * ns
    sq_ref[...] = out[:, HID:]


def _kb_body(a_ref, ds_ref, dd_ref, b0_ref, w_ref, h_ref, t12_ref):
    ns, nd = _norms(ds_ref[...], dd_ref[...])
    h = jnp.maximum(a_ref[...] * nd + b0_ref[...], 0.0)
    h_ref[...] = h
    t12_ref[...] = jnp.dot(h * ns, w_ref[...],
                           preferred_element_type=jnp.float32)


def _kc_body(a_ref, ds_ref, dd_ref, b_ref, noise_ref, z_ref):
    _, nd = _norms(ds_ref[...], dd_ref[...])
    r = a_ref[...] * nd + b_ref[...]
    mean = jnp.maximum(r[:, :OUT], 0.0)
    log_std = r[:, OUT:]
    z_ref[...] = mean + noise_ref[...] * jnp.exp(log_std)


def _row_spec(width):
    return pl.BlockSpec((BLK, width), lambda i: (i, 0))


def _full_spec(shape):
    return pl.BlockSpec(shape, lambda i: (0,) * len(shape))


_ka = pl.pallas_call(
    _ka_body,
    grid=(GRID,),
    in_specs=[_row_spec(IN_DIM), _full_spec((IN_DIM, HID + IN_DIM)),
              _row_spec(1), _row_spec(1)],
    out_specs=[_row_spec(HID), _row_spec(IN_DIM)],
    out_shape=[jax.ShapeDtypeStruct((NPAD, HID), jnp.float32),
               jax.ShapeDtypeStruct((NPAD, IN_DIM), jnp.float32)],
)

_kb = pl.pallas_call(
    _kb_body,
    grid=(GRID,),
    in_specs=[_row_spec(HID), _row_spec(1), _row_spec(1),
              _full_spec((1, HID)), _full_spec((HID, 2 * OUT))],
    out_specs=[_row_spec(HID), _row_spec(2 * OUT)],
    out_shape=[jax.ShapeDtypeStruct((NPAD, HID), jnp.float32),
               jax.ShapeDtypeStruct((NPAD, 2 * OUT), jnp.float32)],
)

_kc = pl.pallas_call(
    _kc_body,
    grid=(GRID,),
    in_specs=[_row_spec(2 * OUT), _row_spec(1), _row_spec(1),
              _full_spec((1, 2 * OUT)), _row_spec(OUT)],
    out_specs=_row_spec(OUT),
    out_shape=jax.ShapeDtypeStruct((NPAD, OUT), jnp.float32),
)


def kernel(features, edge_index, W0, b0, W1, b1, W2, b2, Wfc):
    src = edge_index[0].reshape(NCHKS, 1, CHKE)
    dst = edge_index[1].reshape(NCHKS, 1, CHKE)

    deg_s, deg_d = _deg(src, dst)
    deg_s = deg_s.reshape(NPAD, 1)
    deg_d = deg_d.reshape(NPAD, 1)

    xp = jnp.zeros((NPAD, IN_DIM), jnp.float32).at[:N].set(features)
    wcat_a = jnp.concatenate([W0, Wfc.T], axis=1)
    t0, sq = _ka(xp, wcat_a, deg_s, deg_d)

    agg0 = _scat(src, dst, t0)
    wcat_12 = jnp.concatenate([W1, W2], axis=1)
    h, t12 = _kb(agg0, deg_s, deg_d, b0.reshape(1, HID), wcat_12)

    agg12 = _scat(src, dst, t12)
    noise = jax.random.normal(jax.random.key(42), (N, OUT), jnp.float32)
    noise_p = jnp.zeros((NPAD, OUT), jnp.float32).at[:N].set(noise)
    bcat = jnp.concatenate([b1, b2]).reshape(1, 2 * OUT)
    z = _kc(agg12, deg_s, deg_d, bcat, noise_p)

    return (z[:N], h[:N], sq[:N])


# speedup vs baseline: 1.9479x; 1.9479x over previous
import jax
import jax.numpy as jnp
from jax import lax
from jax.experimental import pallas as pl
from jax.experimental.pallas import tpu as pltpu

N = 10000
E = 320000
IN_DIM = 128
HID = 128
OUT = 64

NPAD = 10240
CHKE = 16000
NCHKS = E // CHKE
BLK = 256
GRID = NPAD // BLK


def _deg_body(src_ref, dst_ref, ds_ref, dd_ref, acs_ref, acd_ref,
              acs2_ref, acd2_ref):
    i = pl.program_id(0)

    @pl.when(i == 0)
    def _():
        def zero(n, _):
            acs_ref[n] = 0.0
            acd_ref[n] = 0.0
            acs2_ref[n] = 0.0
            acd2_ref[n] = 0.0
            return 0
        lax.fori_loop(0, NPAD, zero, 0)

    def body(e, _):
        s0 = src_ref[0, 0, 2 * e]
        d0 = dst_ref[0, 0, 2 * e]
        s1 = src_ref[0, 0, 2 * e + 1]
        d1 = dst_ref[0, 0, 2 * e + 1]
        acs_ref[s0] += 1.0
        acd_ref[d0] += 1.0
        acs2_ref[s1] += 1.0
        acd2_ref[d1] += 1.0
        return 0

    lax.fori_loop(0, CHKE // 2, body, 0)

    @pl.when(i == NCHKS - 1)
    def _():
        def emit(n, _):
            ds_ref[n] = acs_ref[n] + acs2_ref[n]
            dd_ref[n] = acd_ref[n] + acd2_ref[n]
            return 0
        lax.fori_loop(0, NPAD, emit, 0)


_deg = pl.pallas_call(
    _deg_body,
    grid=(NCHKS,),
    in_specs=[pl.BlockSpec((1, 1, CHKE), lambda i: (i, 0, 0),
                           memory_space=pltpu.SMEM),
              pl.BlockSpec((1, 1, CHKE), lambda i: (i, 0, 0),
                           memory_space=pltpu.SMEM)],
    out_specs=[pl.BlockSpec((NPAD,), lambda i: (0,),
                            memory_space=pltpu.SMEM),
               pl.BlockSpec((NPAD,), lambda i: (0,),
                            memory_space=pltpu.SMEM)],
    out_shape=[jax.ShapeDtypeStruct((NPAD,), jnp.float32),
               jax.ShapeDtypeStruct((NPAD,), jnp.float32)],
    scratch_shapes=[pltpu.SMEM((NPAD,), jnp.float32),
                    pltpu.SMEM((NPAD,), jnp.float32),
                    pltpu.SMEM((NPAD,), jnp.float32),
                    pltpu.SMEM((NPAD,), jnp.float32)],
)


def _scat_body(src_ref, dst_ref, t_ref, a_ref, b_ref, c_ref, e_ref):
    i = pl.program_id(0)

    @pl.when(i == 0)
    def _():
        a_ref[...] = jnp.zeros((NPAD, HID), jnp.float32)
        b_ref[...] = jnp.zeros((NPAD, HID), jnp.float32)
        c_ref[...] = jnp.zeros((NPAD, HID), jnp.float32)
        e_ref[...] = jnp.zeros((NPAD, HID), jnp.float32)

    def body(e, _):
        s0 = src_ref[0, 0, 4 * e]
        d0 = dst_ref[0, 0, 4 * e]
        s1 = src_ref[0, 0, 4 * e + 1]
        d1 = dst_ref[0, 0, 4 * e + 1]
        s2 = src_ref[0, 0, 4 * e + 2]
        d2 = dst_ref[0, 0, 4 * e + 2]
        s3 = src_ref[0, 0, 4 * e + 3]
        d3 = dst_ref[0, 0, 4 * e + 3]
        a_ref[pl.ds(d0, 1), :] += t_ref[pl.ds(s0, 1), :]
        b_ref[pl.ds(d1, 1), :] += t_ref[pl.ds(s1, 1), :]
        c_ref[pl.ds(d2, 1), :] += t_ref[pl.ds(s2, 1), :]
        e_ref[pl.ds(d3, 1), :] += t_ref[pl.ds(s3, 1), :]
        return 0

    lax.fori_loop(0, CHKE // 4, body, 0)


_scat = pl.pallas_call(
    _scat_body,
    grid=(NCHKS,),
    in_specs=[pl.BlockSpec((1, 1, CHKE), lambda i: (i, 0, 0),
                           memory_space=pltpu.SMEM),
              pl.BlockSpec((1, 1, CHKE), lambda i: (i, 0, 0),
                           memory_space=pltpu.SMEM),
              pl.BlockSpec((NPAD, HID), lambda i: (0, 0))],
    out_specs=[pl.BlockSpec((NPAD, HID), lambda i: (0, 0))] * 4,
    out_shape=[jax.ShapeDtypeStruct((NPAD, HID), jnp.float32)] * 4,
)


def _norms(ds, dd):
    ns = lax.rsqrt(jnp.maximum(ds, 1.0))
    nd = lax.rsqrt(jnp.maximum(dd, 1.0))
    return ns, nd


def _ka_body(x_ref, w_ref, ds_ref, dd_ref, t0_ref, sq_ref):
    out = jnp.dot(x_ref[...], w_ref[...], preferred_element_type=jnp.float32)
    ns, _ = _norms(ds_ref[...], dd_ref[...])
    t0_ref[...] = out[:, :HID] * ns
    sq_ref[...] = out[:, HID:]


def _kb_body(a_ref, a2_ref, a3_ref, a4_ref, ds_ref, dd_ref, b0_ref, w_ref,
             h_ref, t12_ref):
    ns, nd = _norms(ds_ref[...], dd_ref[...])
    agg = a_ref[...] + a2_ref[...] + a3_ref[...] + a4_ref[...]
    h = jnp.maximum(agg * nd + b0_ref[...], 0.0)
    h_ref[...] = h
    t12_ref[...] = jnp.dot(h * ns, w_ref[...],
                           preferred_element_type=jnp.float32)


def _kc_body(a_ref, a2_ref, a3_ref, a4_ref, ds_ref, dd_ref, b_ref,
             noise_ref, z_ref):
    _, nd = _norms(ds_ref[...], dd_ref[...])
    agg = a_ref[...] + a2_ref[...] + a3_ref[...] + a4_ref[...]
    r = agg * nd + b_ref[...]
    mean = jnp.maximum(r[:, :OUT], 0.0)
    log_std = r[:, OUT:]
    z_ref[...] = mean + noise_ref[...] * jnp.exp(log_std)


def _row_spec(width):
    return pl.BlockSpec((BLK, width), lambda i: (i, 0))


def _full_spec(shape):
    return pl.BlockSpec(shape, lambda i: (0,) * len(shape))


_ka = pl.pallas_call(
    _ka_body,
    grid=(GRID,),
    in_specs=[_row_spec(IN_DIM), _full_spec((IN_DIM, HID + IN_DIM)),
              _row_spec(1), _row_spec(1)],
    out_specs=[_row_spec(HID), _row_spec(IN_DIM)],
    out_shape=[jax.ShapeDtypeStruct((NPAD, HID), jnp.float32),
               jax.ShapeDtypeStruct((NPAD, IN_DIM), jnp.float32)],
)

_kb = pl.pallas_call(
    _kb_body,
    grid=(GRID,),
    in_specs=[_row_spec(HID), _row_spec(HID), _row_spec(HID),
              _row_spec(HID), _row_spec(1), _row_spec(1),
              _full_spec((1, HID)), _full_spec((HID, 2 * OUT))],
    out_specs=[_row_spec(HID), _row_spec(2 * OUT)],
    out_shape=[jax.ShapeDtypeStruct((NPAD, HID), jnp.float32),
               jax.ShapeDtypeStruct((NPAD, 2 * OUT), jnp.float32)],
)

_kc = pl.pallas_call(
    _kc_body,
    grid=(GRID,),
    in_specs=[_row_spec(2 * OUT), _row_spec(2 * OUT), _row_spec(2 * OUT),
              _row_spec(2 * OUT), _row_spec(1), _row_spec(1),
              _full_spec((1, 2 * OUT)), _row_spec(OUT)],
    out_specs=_row_spec(OUT),
    out_shape=jax.ShapeDtypeStruct((NPAD, OUT), jnp.float32),
)


def kernel(features, edge_index, W0, b0, W1, b1, W2, b2, Wfc):
    src = edge_index[0].reshape(NCHKS, 1, CHKE)
    dst = edge_index[1].reshape(NCHKS, 1, CHKE)

    deg_s, deg_d = _deg(src, dst)
    deg_s = deg_s.reshape(NPAD, 1)
    deg_d = deg_d.reshape(NPAD, 1)

    xp = jnp.zeros((NPAD, IN_DIM), jnp.float32).at[:N].set(features)
    wcat_a = jnp.concatenate([W0, Wfc.T], axis=1)
    t0, sq = _ka(xp, wcat_a, deg_s, deg_d)

    g1, g2, g3, g4 = _scat(src, dst, t0)
    wcat_12 = jnp.concatenate([W1, W2], axis=1)
    h, t12 = _kb(g1, g2, g3, g4, deg_s, deg_d, b0.reshape(1, HID), wcat_12)

    m1, m2, m3, m4 = _scat(src, dst, t12)
    noise = jax.random.normal(jax.random.key(42), (N, OUT), jnp.float32)
    noise_p = jnp.zeros((NPAD, OUT), jnp.float32).at[:N].set(noise)
    bcat = jnp.concatenate([b1, b2]).reshape(1, 2 * OUT)
    z = _kc(m1, m2, m3, m4, deg_s, deg_d, bcat, noise_p)

    return (z[:N], h[:N], sq[:N])


# speedup vs baseline: 2.1496x; 1.1036x over previous
import jax
import jax.numpy as jnp
from jax import lax
from jax.experimental import pallas as pl
from jax.experimental.pallas import tpu as pltpu

N = 10000
E = 320000
IN_DIM = 128
HID = 128
OUT = 64

NPAD = 10240
CHKE = 16000
NCHKS = E // CHKE
BLK = 256
GRID = NPAD // BLK


def _deg_body(src_ref, dst_ref, ds_ref, dd_ref, acs_ref, acd_ref,
              acs2_ref, acd2_ref):
    i = pl.program_id(0)

    @pl.when(i == 0)
    def _():
        def zero(n, _):
            acs_ref[n] = 0.0
            acd_ref[n] = 0.0
            acs2_ref[n] = 0.0
            acd2_ref[n] = 0.0
            return 0
        lax.fori_loop(0, NPAD, zero, 0)

    def body(e, _):
        s0 = src_ref[0, 0, 2 * e]
        d0 = dst_ref[0, 0, 2 * e]
        s1 = src_ref[0, 0, 2 * e + 1]
        d1 = dst_ref[0, 0, 2 * e + 1]
        acs_ref[s0] += 1.0
        acd_ref[d0] += 1.0
        acs2_ref[s1] += 1.0
        acd2_ref[d1] += 1.0
        return 0

    lax.fori_loop(0, CHKE // 2, body, 0)

    @pl.when(i == NCHKS - 1)
    def _():
        def emit(n, _):
            ds_ref[n] = acs_ref[n] + acs2_ref[n]
            dd_ref[n] = acd_ref[n] + acd2_ref[n]
            return 0
        lax.fori_loop(0, NPAD, emit, 0)


_deg = pl.pallas_call(
    _deg_body,
    grid=(NCHKS,),
    in_specs=[pl.BlockSpec((1, 1, CHKE), lambda i: (i, 0, 0),
                           memory_space=pltpu.SMEM),
              pl.BlockSpec((1, 1, CHKE), lambda i: (i, 0, 0),
                           memory_space=pltpu.SMEM)],
    out_specs=[pl.BlockSpec((NPAD,), lambda i: (0,),
                            memory_space=pltpu.SMEM),
               pl.BlockSpec((NPAD,), lambda i: (0,),
                            memory_space=pltpu.SMEM)],
    out_shape=[jax.ShapeDtypeStruct((NPAD,), jnp.float32),
               jax.ShapeDtypeStruct((NPAD,), jnp.float32)],
    scratch_shapes=[pltpu.SMEM((NPAD,), jnp.float32),
                    pltpu.SMEM((NPAD,), jnp.float32),
                    pltpu.SMEM((NPAD,), jnp.float32),
                    pltpu.SMEM((NPAD,), jnp.float32)],
)


def _scat_body(src_ref, dst_ref, t_ref, *acc_refs):
    i = pl.program_id(0)

    @pl.when(i == 0)
    def _():
        for r in acc_refs:
            r[...] = jnp.zeros((NPAD, HID), jnp.float32)

    def body(e, _):
        for q, r in enumerate(acc_refs):
            sq = src_ref[0, 0, 8 * e + q]
            dq = dst_ref[0, 0, 8 * e + q]
            r[pl.ds(dq, 1), :] += t_ref[pl.ds(sq, 1), :]
        return 0

    lax.fori_loop(0, CHKE // 8, body, 0)


_scat = pl.pallas_call(
    _scat_body,
    grid=(NCHKS,),
    in_specs=[pl.BlockSpec((1, 1, CHKE), lambda i: (i, 0, 0),
                           memory_space=pltpu.SMEM),
              pl.BlockSpec((1, 1, CHKE), lambda i: (i, 0, 0),
                           memory_space=pltpu.SMEM),
              pl.BlockSpec((NPAD, HID), lambda i: (0, 0))],
    out_specs=[pl.BlockSpec((NPAD, HID), lambda i: (0, 0))] * 8,
    out_shape=[jax.ShapeDtypeStruct((NPAD, HID), jnp.float32)] * 8,
)


def _norms(ds, dd):
    ns = lax.rsqrt(jnp.maximum(ds, 1.0))
    nd = lax.rsqrt(jnp.maximum(dd, 1.0))
    return ns, nd


def _ka_body(x_ref, w_ref, ds_ref, dd_ref, t0_ref, sq_ref):
    out = jnp.dot(x_ref[...], w_ref[...], preferred_element_type=jnp.float32)
    ns, _ = _norms(ds_ref[...], dd_ref[...])
    t0_ref[...] = out[:, :HID] * ns
    sq_ref[...] = out[:, HID:]


def _kb_body(a1, a2, a3, a4, a5, a6, a7, a8, ds_ref, dd_ref, b0_ref, w_ref,
             h_ref, t12_ref):
    ns, nd = _norms(ds_ref[...], dd_ref[...])
    agg = ((a1[...] + a2[...]) + (a3[...] + a4[...])) + (
        (a5[...] + a6[...]) + (a7[...] + a8[...]))
    h = jnp.maximum(agg * nd + b0_ref[...], 0.0)
    h_ref[...] = h
    t12_ref[...] = jnp.dot(h * ns, w_ref[...],
                           preferred_element_type=jnp.float32)


def _kc_body(a1, a2, a3, a4, a5, a6, a7, a8, ds_ref, dd_ref, b_ref,
             noise_ref, z_ref):
    _, nd = _norms(ds_ref[...], dd_ref[...])
    agg = ((a1[...] + a2[...]) + (a3[...] + a4[...])) + (
        (a5[...] + a6[...]) + (a7[...] + a8[...]))
    r = agg * nd + b_ref[...]
    mean = jnp.maximum(r[:, :OUT], 0.0)
    log_std = r[:, OUT:]
    z_ref[...] = mean + noise_ref[...] * jnp.exp(log_std)


def _row_spec(width):
    return pl.BlockSpec((BLK, width), lambda i: (i, 0))


def _full_spec(shape):
    return pl.BlockSpec(shape, lambda i: (0,) * len(shape))


_ka = pl.pallas_call(
    _ka_body,
    grid=(GRID,),
    in_specs=[_row_spec(IN_DIM), _full_spec((IN_DIM, HID + IN_DIM)),
              _row_spec(1), _row_spec(1)],
    out_specs=[_row_spec(HID), _row_spec(IN_DIM)],
    out_shape=[jax.ShapeDtypeStruct((NPAD, HID), jnp.float32),
               jax.ShapeDtypeStruct((NPAD, IN_DIM), jnp.float32)],
)

_kb = pl.pallas_call(
    _kb_body,
    grid=(GRID,),
    in_specs=[_row_spec(HID)] * 8 + [_row_spec(1), _row_spec(1),
              _full_spec((1, HID)), _full_spec((HID, 2 * OUT))],
    out_specs=[_row_spec(HID), _row_spec(2 * OUT)],
    out_shape=[jax.ShapeDtypeStruct((NPAD, HID), jnp.float32),
               jax.ShapeDtypeStruct((NPAD, 2 * OUT), jnp.float32)],
)

_kc = pl.pallas_call(
    _kc_body,
    grid=(GRID,),
    in_specs=[_row_spec(2 * OUT)] * 8 + [_row_spec(1), _row_spec(1),
              _full_spec((1, 2 * OUT)), _row_spec(OUT)],
    out_specs=_row_spec(OUT),
    out_shape=jax.ShapeDtypeStruct((NPAD, OUT), jnp.float32),
)


def kernel(features, edge_index, W0, b0, W1, b1, W2, b2, Wfc):
    src = edge_index[0].reshape(NCHKS, 1, CHKE)
    dst = edge_index[1].reshape(NCHKS, 1, CHKE)

    deg_s, deg_d = _deg(src, dst)
    deg_s = deg_s.reshape(NPAD, 1)
    deg_d = deg_d.reshape(NPAD, 1)

    xp = jnp.zeros((NPAD, IN_DIM), jnp.float32).at[:N].set(features)
    wcat_a = jnp.concatenate([W0, Wfc.T], axis=1)
    t0, sq = _ka(xp, wcat_a, deg_s, deg_d)

    gs = _scat(src, dst, t0)
    wcat_12 = jnp.concatenate([W1, W2], axis=1)
    h, t12 = _kb(*gs, deg_s, deg_d, b0.reshape(1, HID), wcat_12)

    ms = _scat(src, dst, t12)
    noise = jax.random.normal(jax.random.key(42), (N, OUT), jnp.float32)
    noise_p = jnp.zeros((NPAD, OUT), jnp.float32).at[:N].set(noise)
    bcat = jnp.concatenate([b1, b2]).reshape(1, 2 * OUT)
    z = _kc(*ms, deg_s, deg_d, bcat, noise_p)

    return (z[:N], h[:N], sq[:N])
